# baseline (speedup 1.0000x reference)
import jax
import jax.numpy as jnp
from jax import lax
from jax.experimental import pallas as pl
from jax.experimental.pallas import tpu as pltpu

U = 196
UP = 208
TOPK = 64
NH = 4
HD = 64
D = NH * HD
PB = 8
RB = 8


def _b16(x):
    return x.astype(jnp.bfloat16)


def _bdot(x, y):
    return jnp.dot(_b16(x), _b16(y), preferred_element_type=jnp.float32)


def _bdot_t(x, y):
    return lax.dot_general(_b16(x), _b16(y), (((1,), (1,)), ((), ())),
                           preferred_element_type=jnp.float32)


def _k1_body(x_ref, w1t_ref, b1_ref, w2t_ref, b2_ref,
             lnkg_ref, lnkb_ref, lnqg_ref, lnqb_ref,
             wkt_ref, bk_ref, wvt_ref, bv_ref, c_ref):
    for i in range(PB):
        x = x_ref[i]
        h = _bdot(x, w1t_ref[...]) + b1_ref[...]
        h = jnp.where(h > 0, h, 0.01 * h)
        a = _bdot(h, w2t_ref[...]) + b2_ref[...]
        anorm = jnp.sqrt(jnp.sum(a * a, axis=1, keepdims=True))
        an = a / jnp.maximum(anorm, 1e-12)
        mu = jnp.mean(a, axis=-1, keepdims=True)
        var = jnp.mean((a - mu) * (a - mu), axis=-1, keepdims=True)
        lnc = (a - mu) / jnp.sqrt(var + 1e-5)
        lnq = lnc * lnqg_ref[...] + lnqb_ref[...]
        lnk = lnc * lnkg_ref[...] + lnkb_ref[...]
        kp = _bdot(lnk, wkt_ref[...]) + bk_ref[...]
        vp = _bdot(lnk, wvt_ref[...]) + bv_ref[...]
        packed = jnp.concatenate(
            [_b16(an), _b16(lnq), _b16(kp), _b16(vp)], axis=1)
        c_ref[i * UP:i * UP + U, :] = packed
        c_ref[i * UP + U:(i + 1) * UP, :] = jnp.zeros(
            (UP - U, 4 * D), jnp.bfloat16)


def _k2_body(ids_ref, idx_sm_ref, idx_ref, *refs):
    c_refs = refs[:RB]
    (wqt_ref, bq_ref, owt_ref, ob_ref, out_ref) = refs[RB:]

    idxv = idx_ref[...]
    rowi = lax.broadcasted_iota(jnp.int32, (RB, 1), 0)
    row32 = lax.broadcasted_iota(jnp.int32, (4 * RB, 1), 0)

    toki = lax.broadcasted_iota(jnp.int32, (RB, UP), 1)
    onehot = _b16((toki == idxv).astype(jnp.float32))

    ql = jnp.zeros((RB, 2 * D), jnp.float32)
    for j in range(RB):
        e = _bdot(onehot, c_refs[j][:, 0:2 * D])
        ql = ql + e * (rowi == j).astype(jnp.float32)
    qn = ql[:, 0:D]
    lnq = ql[:, D:2 * D]

    sim = jnp.zeros((RB, UP), jnp.float32)
    for j in range(RB):
        s = _bdot_t(qn, c_refs[j][:, 0:D])
        sim = sim + s * (rowi == j).astype(jnp.float32)
    col = lax.broadcasted_iota(jnp.int32, (RB, UP), 1)
    simv = jnp.where(col < U, sim, -3.0)

    key = lax.bitcast_convert_type(simv, jnp.int32)
    key = jnp.where(key < 0, key ^ jnp.int32(0x7FFFFFFF), key)
    int_min = jnp.int32(-2147483648)
    p = jnp.zeros((RB, 1), jnp.int32)
    for it in range(8):
        shift = 28 - 4 * it
        inds = jnp.zeros((RB, 1), jnp.int32)
        for v in range(1, 16):
            c = v << shift
            if c >= 1 << 31:
                c -= 1 << 32
            test = p + jnp.int32(c)
            cnt = jnp.sum((key >= (test ^ int_min)).astype(jnp.int32),
                          axis=1, keepdims=True)
            inds = inds + (cnt >= TOPK).astype(jnp.int32)
        p = p + (inds << shift)
    selected = key >= (p ^ int_min)

    qp = _bdot(lnq, wqt_ref[...]) + bq_ref[...]

    pr = lax.broadcasted_iota(jnp.int32, (4 * RB, RB), 0)
    pc = lax.broadcasted_iota(jnp.int32, (4 * RB, RB), 1)
    psel = _b16((pr // NH == pc).astype(jnp.float32))
    hr = lax.broadcasted_iota(jnp.int32, (4 * RB, D), 0)
    hc = lax.broadcasted_iota(jnp.int32, (4 * RB, D), 1)
    hmask = (hc // HD == hr % NH).astype(jnp.float32)
    p2r = lax.broadcasted_iota(jnp.int32, (RB, 4 * RB), 0)
    p2c = lax.broadcasted_iota(jnp.int32, (RB, 4 * RB), 1)
    p2 = _b16((p2c // NH == p2r).astype(jnp.float32))

    qh32 = _bdot(psel, qp) * hmask
    sel32 = _bdot(psel, selected.astype(jnp.float32)) > 0.5

    scale = 1.0 / jnp.sqrt(jnp.float32(HD))
    s32 = jnp.zeros((4 * RB, UP), jnp.float32)
    for j in range(RB):
        sj = _bdot_t(qh32, c_refs[j][:, 2 * D:3 * D])
        s32 = s32 + sj * (row32 // NH == j).astype(jnp.float32)
    s32 = s32 * scale
    s32 = jnp.where(sel32, s32, -1e30)
    mx = jnp.max(s32, axis=1, keepdims=True)
    e32 = jnp.where(sel32, jnp.exp(s32 - mx), 0.0)
    attn32 = e32 / jnp.sum(e32, axis=1, keepdims=True)

    ctx32 = jnp.zeros((4 * RB, D), jnp.float32)
    for j in range(RB):
        aj = attn32 * (row32 // NH == j).astype(jnp.float32)
        ctx32 = ctx32 + _bdot(aj, c_refs[j][:, 3 * D:4 * D])
    ctx = _bdot(p2, ctx32 * hmask)

    out_ref[...] = _bdot(ctx, owt_ref[...]) + ob_ref[...]


def kernel(patches, patch_ids, valid_mask, patch_center_gps, offsets,
           W1, b1, W2, b2, lnq_g, lnq_b, lnk_g, lnk_b, in_w, in_b, out_w, out_b):
    N, u, Din = patches.shape
    M = patch_ids.shape[0]
    hid = W1.shape[0]

    hg = int(u ** 0.5)
    dx = offsets[:, 0]
    dy = offsets[:, 1]
    i_t = jnp.clip(hg // 2 + dy, 0, hg - 1)
    j_t = jnp.clip(hg // 2 + dx, 0, hg - 1)
    idx_flat = (i_t * hg + j_t).astype(jnp.int32)
    ids = patch_ids.astype(jnp.int32)

    wq, wk, wv = in_w[:D], in_w[D:2 * D], in_w[2 * D:]
    bq, bk, bv = in_b[:D], in_b[D:2 * D], in_b[2 * D:]
    row2 = lambda v: v.reshape(1, -1)

    full2 = lambda r, c: pl.BlockSpec((r, c), lambda t: (0, 0))
    c_all = pl.pallas_call(
        _k1_body,
        grid=(N // PB,),
        in_specs=[
            pl.BlockSpec((PB, u, Din), lambda t: (t, 0, 0)),
            full2(Din, hid), full2(1, hid),
            full2(hid, D), full2(1, D),
            full2(1, D), full2(1, D), full2(1, D), full2(1, D),
            full2(D, D), full2(1, D),
            full2(D, D), full2(1, D),
        ],
        out_specs=pl.BlockSpec((PB * UP, 4 * D), lambda t: (t, 0)),
        out_shape=jax.ShapeDtypeStruct((N * UP, 4 * D), jnp.bfloat16),
    )(patches, W1.T, row2(b1), W2.T, row2(b2),
      row2(lnk_g), row2(lnk_b), row2(lnq_g), row2(lnq_b),
      wk.T, row2(bk), wv.T, row2(bv))

    def gat(j):
        return pl.BlockSpec(
            (UP, 4 * D), lambda m, ids_r, idx_r, j=j: (ids_r[m * RB + j], 0))
    cst = lambda r, c: pl.BlockSpec((r, c), lambda m, ids_r, idx_r: (0, 0))
    grid_spec = pltpu.PrefetchScalarGridSpec(
        num_scalar_prefetch=2,
        grid=(M // RB,),
        in_specs=(
            [pl.BlockSpec((RB, 1), lambda m, ids_r, idx_r: (m, 0))] +
            [gat(j) for j in range(RB)] +
            [cst(D, D), cst(1, D), cst(D, D), cst(1, D)]
        ),
        out_specs=pl.BlockSpec((RB, D), lambda m, ids_r, idx_r: (m, 0)),
    )
    attn_out = pl.pallas_call(
        _k2_body,
        grid_spec=grid_spec,
        out_shape=jax.ShapeDtypeStruct((M, D), jnp.float32),
    )(ids, idx_flat, idx_flat.reshape(M, 1), *([c_all] * RB),
      wq.T, row2(bq), out_w.T, row2(out_b))

    B, T = valid_mask.shape
    flat_mask = valid_mask.reshape(-1)
    rank = jnp.cumsum(flat_mask.astype(jnp.int32)) - 1
    placed = attn_out[jnp.clip(rank, 0, M - 1)]
    return jnp.where(flat_mask[:, None], placed,
                     jnp.zeros((), dtype=attn_out.dtype)).reshape(B, T, D)


# speedup vs baseline: 1.0988x; 1.0988x over previous
import jax
import jax.numpy as jnp
from jax import lax
from jax.experimental import pallas as pl
from jax.experimental.pallas import tpu as pltpu

U = 196
UP = 208
TOPK = 64
NH = 4
HD = 64
D = NH * HD
PB = 16
RB = 16


def _b16(x):
    return x.astype(jnp.bfloat16)


def _bdot(x, y):
    return jnp.dot(_b16(x), _b16(y), preferred_element_type=jnp.float32)


def _bdot_t(x, y):
    return lax.dot_general(_b16(x), _b16(y), (((1,), (1,)), ((), ())),
                           preferred_element_type=jnp.float32)


def _k1_body(x_ref, w1t_ref, b1_ref, w2t_ref, b2_ref,
             lnkg_ref, lnkb_ref, lnqg_ref, lnqb_ref,
             wkt_ref, bk_ref, wvt_ref, bv_ref, c_ref):
    for i in range(PB):
        x = x_ref[i]
        h = _bdot(x, w1t_ref[...]) + b1_ref[...]
        h = jnp.where(h > 0, h, 0.01 * h)
        a = _bdot(h, w2t_ref[...]) + b2_ref[...]
        anorm = jnp.sqrt(jnp.sum(a * a, axis=1, keepdims=True))
        an = a / jnp.maximum(anorm, 1e-12)
        mu = jnp.mean(a, axis=-1, keepdims=True)
        var = jnp.mean((a - mu) * (a - mu), axis=-1, keepdims=True)
        lnc = (a - mu) / jnp.sqrt(var + 1e-5)
        lnq = lnc * lnqg_ref[...] + lnqb_ref[...]
        lnk = lnc * lnkg_ref[...] + lnkb_ref[...]
        kp = _bdot(lnk, wkt_ref[...]) + bk_ref[...]
        vp = _bdot(lnk, wvt_ref[...]) + bv_ref[...]
        packed = jnp.concatenate(
            [_b16(an), _b16(lnq), _b16(kp), _b16(vp)], axis=1)
        c_ref[i * UP:i * UP + U, :] = packed
        c_ref[i * UP + U:(i + 1) * UP, :] = jnp.zeros(
            (UP - U, 4 * D), jnp.bfloat16)


def _k2_body(ids_ref, idx_sm_ref, idx_ref, *refs):
    c_refs = refs[:RB]
    (wqt_ref, bq_ref, owt_ref, ob_ref, out_ref) = refs[RB:]

    idxv = idx_ref[...]
    rowi = lax.broadcasted_iota(jnp.int32, (RB, 1), 0)
    row32 = lax.broadcasted_iota(jnp.int32, (4 * RB, 1), 0)

    toki = lax.broadcasted_iota(jnp.int32, (RB, UP), 1)
    onehot = _b16((toki == idxv).astype(jnp.float32))

    ql = jnp.zeros((RB, 2 * D), jnp.float32)
    for j in range(RB):
        e = _bdot(onehot, c_refs[j][:, 0:2 * D])
        ql = ql + e * (rowi == j).astype(jnp.float32)
    qn = ql[:, 0:D]
    lnq = ql[:, D:2 * D]

    sim = jnp.zeros((RB, UP), jnp.float32)
    for j in range(RB):
        s = _bdot_t(qn, c_refs[j][:, 0:D])
        sim = sim + s * (rowi == j).astype(jnp.float32)
    col = lax.broadcasted_iota(jnp.int32, (RB, UP), 1)
    simv = jnp.where(col < U, sim, -3.0)

    key = lax.bitcast_convert_type(simv, jnp.int32)
    key = jnp.where(key < 0, key ^ jnp.int32(0x7FFFFFFF), key)
    int_min = jnp.int32(-2147483648)
    p = jnp.zeros((RB, 1), jnp.int32)
    for it in range(8):
        shift = 28 - 4 * it
        inds = jnp.zeros((RB, 1), jnp.int32)
        for v in range(1, 16):
            c = v << shift
            if c >= 1 << 31:
                c -= 1 << 32
            test = p + jnp.int32(c)
            cnt = jnp.sum((key >= (test ^ int_min)).astype(jnp.int32),
                          axis=1, keepdims=True)
            inds = inds + (cnt >= TOPK).astype(jnp.int32)
        p = p + (inds << shift)
    selected = key >= (p ^ int_min)

    qp = _bdot(lnq, wqt_ref[...]) + bq_ref[...]

    pr = lax.broadcasted_iota(jnp.int32, (4 * RB, RB), 0)
    pc = lax.broadcasted_iota(jnp.int32, (4 * RB, RB), 1)
    psel = _b16((pr // NH == pc).astype(jnp.float32))
    hr = lax.broadcasted_iota(jnp.int32, (4 * RB, D), 0)
    hc = lax.broadcasted_iota(jnp.int32, (4 * RB, D), 1)
    hmask = (hc // HD == hr % NH).astype(jnp.float32)
    p2r = lax.broadcasted_iota(jnp.int32, (RB, 4 * RB), 0)
    p2c = lax.broadcasted_iota(jnp.int32, (RB, 4 * RB), 1)
    p2 = _b16((p2c // NH == p2r).astype(jnp.float32))

    qh32 = _bdot(psel, qp) * hmask
    sel32 = _bdot(psel, selected.astype(jnp.float32)) > 0.5

    scale = 1.0 / jnp.sqrt(jnp.float32(HD))
    s32 = jnp.zeros((4 * RB, UP), jnp.float32)
    for j in range(RB):
        sj = _bdot_t(qh32, c_refs[j][:, 2 * D:3 * D])
        s32 = s32 + sj * (row32 // NH == j).astype(jnp.float32)
    s32 = s32 * scale
    s32 = jnp.where(sel32, s32, -1e30)
    mx = jnp.max(s32, axis=1, keepdims=True)
    e32 = jnp.where(sel32, jnp.exp(s32 - mx), 0.0)
    attn32 = e32 / jnp.sum(e32, axis=1, keepdims=True)

    ctx32 = jnp.zeros((4 * RB, D), jnp.float32)
    for j in range(RB):
        aj = attn32 * (row32 // NH == j).astype(jnp.float32)
        ctx32 = ctx32 + _bdot(aj, c_refs[j][:, 3 * D:4 * D])
    ctx = _bdot(p2, ctx32 * hmask)

    out_ref[...] = _bdot(ctx, owt_ref[...]) + ob_ref[...]


def kernel(patches, patch_ids, valid_mask, patch_center_gps, offsets,
           W1, b1, W2, b2, lnq_g, lnq_b, lnk_g, lnk_b, in_w, in_b, out_w, out_b):
    N, u, Din = patches.shape
    M = patch_ids.shape[0]
    hid = W1.shape[0]

    hg = int(u ** 0.5)
    dx = offsets[:, 0]
    dy = offsets[:, 1]
    i_t = jnp.clip(hg // 2 + dy, 0, hg - 1)
    j_t = jnp.clip(hg // 2 + dx, 0, hg - 1)
    idx_flat = (i_t * hg + j_t).astype(jnp.int32)
    ids = patch_ids.astype(jnp.int32)

    wq, wk, wv = in_w[:D], in_w[D:2 * D], in_w[2 * D:]
    bq, bk, bv = in_b[:D], in_b[D:2 * D], in_b[2 * D:]
    row2 = lambda v: v.reshape(1, -1)

    full2 = lambda r, c: pl.BlockSpec((r, c), lambda t: (0, 0))
    c_all = pl.pallas_call(
        _k1_body,
        grid=(N // PB,),
        in_specs=[
            pl.BlockSpec((PB, u, Din), lambda t: (t, 0, 0)),
            full2(Din, hid), full2(1, hid),
            full2(hid, D), full2(1, D),
            full2(1, D), full2(1, D), full2(1, D), full2(1, D),
            full2(D, D), full2(1, D),
            full2(D, D), full2(1, D),
        ],
        out_specs=pl.BlockSpec((PB * UP, 4 * D), lambda t: (t, 0)),
        out_shape=jax.ShapeDtypeStruct((N * UP, 4 * D), jnp.bfloat16),
    )(patches, W1.T, row2(b1), W2.T, row2(b2),
      row2(lnk_g), row2(lnk_b), row2(lnq_g), row2(lnq_b),
      wk.T, row2(bk), wv.T, row2(bv))

    def gat(j):
        return pl.BlockSpec(
            (UP, 4 * D), lambda m, ids_r, idx_r, j=j: (ids_r[m * RB + j], 0))
    cst = lambda r, c: pl.BlockSpec((r, c), lambda m, ids_r, idx_r: (0, 0))
    grid_spec = pltpu.PrefetchScalarGridSpec(
        num_scalar_prefetch=2,
        grid=(M // RB,),
        in_specs=(
            [pl.BlockSpec((RB, 1), lambda m, ids_r, idx_r: (m, 0))] +
            [gat(j) for j in range(RB)] +
            [cst(D, D), cst(1, D), cst(D, D), cst(1, D)]
        ),
        out_specs=pl.BlockSpec((RB, D), lambda m, ids_r, idx_r: (m, 0)),
    )
    attn_out = pl.pallas_call(
        _k2_body,
        grid_spec=grid_spec,
        out_shape=jax.ShapeDtypeStruct((M, D), jnp.float32),
    )(ids, idx_flat, idx_flat.reshape(M, 1), *([c_all] * RB),
      wq.T, row2(bq), out_w.T, row2(out_b))

    B, T = valid_mask.shape
    flat_mask = valid_mask.reshape(-1)
    rank = jnp.cumsum(flat_mask.astype(jnp.int32)) - 1
    placed = attn_out[jnp.clip(rank, 0, M - 1)]
    return jnp.where(flat_mask[:, None], placed,
                     jnp.zeros((), dtype=attn_out.dtype)).reshape(B, T, D)


# speedup vs baseline: 1.1168x; 1.0163x over previous
import jax
import jax.numpy as jnp
from jax import lax
from jax.experimental import pallas as pl
from jax.experimental.pallas import tpu as pltpu

U = 196
UP = 208
TOPK = 64
NH = 4
HD = 64
D = NH * HD
PB = 32
RB = 32


def _b16(x):
    return x.astype(jnp.bfloat16)


def _bdot(x, y):
    return jnp.dot(_b16(x), _b16(y), preferred_element_type=jnp.float32)


def _bdot_t(x, y):
    return lax.dot_general(_b16(x), _b16(y), (((1,), (1,)), ((), ())),
                           preferred_element_type=jnp.float32)


def _k1_body(x_ref, w1t_ref, b1_ref, w2t_ref, b2_ref,
             lnkg_ref, lnkb_ref, lnqg_ref, lnqb_ref,
             wkt_ref, bk_ref, wvt_ref, bv_ref, c_ref):
    for i in range(PB):
        x = x_ref[i]
        h = _bdot(x, w1t_ref[...]) + b1_ref[...]
        h = jnp.where(h > 0, h, 0.01 * h)
        a = _bdot(h, w2t_ref[...]) + b2_ref[...]
        anorm = jnp.sqrt(jnp.sum(a * a, axis=1, keepdims=True))
        an = a / jnp.maximum(anorm, 1e-12)
        mu = jnp.mean(a, axis=-1, keepdims=True)
        var = jnp.mean((a - mu) * (a - mu), axis=-1, keepdims=True)
        lnc = (a - mu) / jnp.sqrt(var + 1e-5)
        lnq = lnc * lnqg_ref[...] + lnqb_ref[...]
        lnk = lnc * lnkg_ref[...] + lnkb_ref[...]
        kp = _bdot(lnk, wkt_ref[...]) + bk_ref[...]
        vp = _bdot(lnk, wvt_ref[...]) + bv_ref[...]
        packed = jnp.concatenate(
            [_b16(an), _b16(lnq), _b16(kp), _b16(vp)], axis=1)
        c_ref[i * UP:i * UP + U, :] = packed
        c_ref[i * UP + U:(i + 1) * UP, :] = jnp.zeros(
            (UP - U, 4 * D), jnp.bfloat16)


def _k2_body(ids_ref, idx_sm_ref, idx_ref, *refs):
    c_refs = refs[:RB]
    (wqt_ref, bq_ref, owt_ref, ob_ref, out_ref) = refs[RB:]

    idxv = idx_ref[...]
    rowi = lax.broadcasted_iota(jnp.int32, (RB, 1), 0)
    row32 = lax.broadcasted_iota(jnp.int32, (4 * RB, 1), 0)

    toki = lax.broadcasted_iota(jnp.int32, (RB, UP), 1)
    onehot = _b16((toki == idxv).astype(jnp.float32))

    ql = jnp.zeros((RB, 2 * D), jnp.float32)
    for j in range(RB):
        e = _bdot(onehot, c_refs[j][:, 0:2 * D])
        ql = ql + e * (rowi == j).astype(jnp.float32)
    qn = ql[:, 0:D]
    lnq = ql[:, D:2 * D]

    sim = jnp.zeros((RB, UP), jnp.float32)
    for j in range(RB):
        s = _bdot_t(qn, c_refs[j][:, 0:D])
        sim = sim + s * (rowi == j).astype(jnp.float32)
    col = lax.broadcasted_iota(jnp.int32, (RB, UP), 1)
    simv = jnp.where(col < U, sim, -3.0)

    key = lax.bitcast_convert_type(simv, jnp.int32)
    key = jnp.where(key < 0, key ^ jnp.int32(0x7FFFFFFF), key)
    int_min = jnp.int32(-2147483648)
    p = jnp.zeros((RB, 1), jnp.int32)
    for it in range(8):
        shift = 28 - 4 * it
        inds = jnp.zeros((RB, 1), jnp.int32)
        for v in range(1, 16):
            c = v << shift
            if c >= 1 << 31:
                c -= 1 << 32
            test = p + jnp.int32(c)
            cnt = jnp.sum((key >= (test ^ int_min)).astype(jnp.int32),
                          axis=1, keepdims=True)
            inds = inds + (cnt >= TOPK).astype(jnp.int32)
        p = p + (inds << shift)
    selected = key >= (p ^ int_min)

    qp = _bdot(lnq, wqt_ref[...]) + bq_ref[...]

    pr = lax.broadcasted_iota(jnp.int32, (4 * RB, RB), 0)
    pc = lax.broadcasted_iota(jnp.int32, (4 * RB, RB), 1)
    psel = _b16((pr // NH == pc).astype(jnp.float32))
    hr = lax.broadcasted_iota(jnp.int32, (4 * RB, D), 0)
    hc = lax.broadcasted_iota(jnp.int32, (4 * RB, D), 1)
    hmask = (hc // HD == hr % NH).astype(jnp.float32)
    p2r = lax.broadcasted_iota(jnp.int32, (RB, 4 * RB), 0)
    p2c = lax.broadcasted_iota(jnp.int32, (RB, 4 * RB), 1)
    p2 = _b16((p2c // NH == p2r).astype(jnp.float32))

    qh32 = _bdot(psel, qp) * hmask
    sel32 = _bdot(psel, selected.astype(jnp.float32)) > 0.5

    scale = 1.0 / jnp.sqrt(jnp.float32(HD))
    s32 = jnp.zeros((4 * RB, UP), jnp.float32)
    for j in range(RB):
        sj = _bdot_t(qh32, c_refs[j][:, 2 * D:3 * D])
        s32 = s32 + sj * (row32 // NH == j).astype(jnp.float32)
    s32 = s32 * scale
    s32 = jnp.where(sel32, s32, -1e30)
    mx = jnp.max(s32, axis=1, keepdims=True)
    e32 = jnp.where(sel32, jnp.exp(s32 - mx), 0.0)
    attn32 = e32 / jnp.sum(e32, axis=1, keepdims=True)

    ctx32 = jnp.zeros((4 * RB, D), jnp.float32)
    for j in range(RB):
        aj = attn32 * (row32 // NH == j).astype(jnp.float32)
        ctx32 = ctx32 + _bdot(aj, c_refs[j][:, 3 * D:4 * D])
    ctx = _bdot(p2, ctx32 * hmask)

    out_ref[...] = _bdot(ctx, owt_ref[...]) + ob_ref[...]


def kernel(patches, patch_ids, valid_mask, patch_center_gps, offsets,
           W1, b1, W2, b2, lnq_g, lnq_b, lnk_g, lnk_b, in_w, in_b, out_w, out_b):
    N, u, Din = patches.shape
    M = patch_ids.shape[0]
    hid = W1.shape[0]

    hg = int(u ** 0.5)
    dx = offsets[:, 0]
    dy = offsets[:, 1]
    i_t = jnp.clip(hg // 2 + dy, 0, hg - 1)
    j_t = jnp.clip(hg // 2 + dx, 0, hg - 1)
    idx_flat = (i_t * hg + j_t).astype(jnp.int32)
    ids = patch_ids.astype(jnp.int32)

    wq, wk, wv = in_w[:D], in_w[D:2 * D], in_w[2 * D:]
    bq, bk, bv = in_b[:D], in_b[D:2 * D], in_b[2 * D:]
    row2 = lambda v: v.reshape(1, -1)

    full2 = lambda r, c: pl.BlockSpec((r, c), lambda t: (0, 0))
    c_all = pl.pallas_call(
        _k1_body,
        grid=(N // PB,),
        in_specs=[
            pl.BlockSpec((PB, u, Din), lambda t: (t, 0, 0)),
            full2(Din, hid), full2(1, hid),
            full2(hid, D), full2(1, D),
            full2(1, D), full2(1, D), full2(1, D), full2(1, D),
            full2(D, D), full2(1, D),
            full2(D, D), full2(1, D),
        ],
        out_specs=pl.BlockSpec((PB * UP, 4 * D), lambda t: (t, 0)),
        out_shape=jax.ShapeDtypeStruct((N * UP, 4 * D), jnp.bfloat16),
    )(patches, W1.T, row2(b1), W2.T, row2(b2),
      row2(lnk_g), row2(lnk_b), row2(lnq_g), row2(lnq_b),
      wk.T, row2(bk), wv.T, row2(bv))

    def gat(j):
        return pl.BlockSpec(
            (UP, 4 * D), lambda m, ids_r, idx_r, j=j: (ids_r[m * RB + j], 0))
    cst = lambda r, c: pl.BlockSpec((r, c), lambda m, ids_r, idx_r: (0, 0))
    grid_spec = pltpu.PrefetchScalarGridSpec(
        num_scalar_prefetch=2,
        grid=(M // RB,),
        in_specs=(
            [pl.BlockSpec((RB, 1), lambda m, ids_r, idx_r: (m, 0))] +
            [gat(j) for j in range(RB)] +
            [cst(D, D), cst(1, D), cst(D, D), cst(1, D)]
        ),
        out_specs=pl.BlockSpec((RB, D), lambda m, ids_r, idx_r: (m, 0)),
    )
    attn_out = pl.pallas_call(
        _k2_body,
        grid_spec=grid_spec,
        out_shape=jax.ShapeDtypeStruct((M, D), jnp.float32),
    )(ids, idx_flat, idx_flat.reshape(M, 1), *([c_all] * RB),
      wq.T, row2(bq), out_w.T, row2(out_b))

    B, T = valid_mask.shape
    flat_mask = valid_mask.reshape(-1)
    rank = jnp.cumsum(flat_mask.astype(jnp.int32)) - 1
    placed = attn_out[jnp.clip(rank, 0, M - 1)]
    return jnp.where(flat_mask[:, None], placed,
                     jnp.zeros((), dtype=attn_out.dtype)).reshape(B, T, D)


# speedup vs baseline: 1.2706x; 1.1377x over previous
import jax
import jax.numpy as jnp
from jax import lax
from jax.experimental import pallas as pl
from jax.experimental.pallas import tpu as pltpu

U = 196
UP = 208
TOPK = 64
NH = 4
HD = 64
D = NH * HD
PB = 16
RB = 32


def _b16(x):
    return x.astype(jnp.bfloat16)


def _bdot(x, y):
    return jnp.dot(_b16(x), _b16(y), preferred_element_type=jnp.float32)


def _bdot_t(x, y):
    return lax.dot_general(_b16(x), _b16(y), (((1,), (1,)), ((), ())),
                           preferred_element_type=jnp.float32)


def _bdot3(x, y):
    return lax.dot_general(_b16(x), _b16(y), (((2,), (0,)), ((), ())),
                           preferred_element_type=jnp.float32)


def _k1_body(x_ref, w1t_ref, b1_ref, w2t_ref, b2_ref,
             lnkg_ref, lnkb_ref, lnqg_ref, lnqb_ref,
             wkt_ref, bk_ref, wvt_ref, bv_ref, c_ref):
    x = x_ref[...]
    h = _bdot3(x, w1t_ref[...]) + b1_ref[...]
    h = jnp.where(h > 0, h, 0.01 * h)
    a = _bdot3(h, w2t_ref[...]) + b2_ref[...]
    anorm = jnp.sqrt(jnp.sum(a * a, axis=2, keepdims=True))
    an = a / jnp.maximum(anorm, 1e-12)
    mu = jnp.mean(a, axis=-1, keepdims=True)
    var = jnp.mean((a - mu) * (a - mu), axis=-1, keepdims=True)
    lnc = (a - mu) / jnp.sqrt(var + 1e-5)
    lnq = lnc * lnqg_ref[...] + lnqb_ref[...]
    lnk = lnc * lnkg_ref[...] + lnkb_ref[...]
    kp = _bdot3(lnk, wkt_ref[...]) + bk_ref[...]
    vp = _bdot3(lnk, wvt_ref[...]) + bv_ref[...]
    packed = jnp.concatenate(
        [_b16(an), _b16(lnq), _b16(kp), _b16(vp)], axis=2)
    for i in range(PB):
        c_ref[i * UP:i * UP + U, :] = packed[i]
        c_ref[i * UP + U:(i + 1) * UP, :] = jnp.zeros(
            (UP - U, 4 * D), jnp.bfloat16)


def _k2_body(ids_ref, idx_sm_ref, idx_ref, *refs):
    c_refs = refs[:RB]
    (wqt_ref, bq_ref, owt_ref, ob_ref, out_ref) = refs[RB:]

    idxv = idx_ref[...]
    rowi = lax.broadcasted_iota(jnp.int32, (RB, 1), 0)
    row32 = lax.broadcasted_iota(jnp.int32, (4 * RB, 1), 0)

    toki = lax.broadcasted_iota(jnp.int32, (RB, UP), 1)
    onehot = _b16((toki == idxv).astype(jnp.float32))

    ql = jnp.zeros((RB, 2 * D), jnp.float32)
    for j in range(RB):
        e = _bdot(onehot, c_refs[j][:, 0:2 * D])
        ql = ql + e * (rowi == j).astype(jnp.float32)
    qn = ql[:, 0:D]
    lnq = ql[:, D:2 * D]

    sim = jnp.zeros((RB, UP), jnp.float32)
    for j in range(RB):
        s = _bdot_t(qn, c_refs[j][:, 0:D])
        sim = sim + s * (rowi == j).astype(jnp.float32)
    col = lax.broadcasted_iota(jnp.int32, (RB, UP), 1)
    simv = jnp.where(col < U, sim, -3.0)

    key = lax.bitcast_convert_type(simv, jnp.int32)
    key = jnp.where(key < 0, key ^ jnp.int32(0x7FFFFFFF), key)
    int_min = jnp.int32(-2147483648)
    p = jnp.zeros((RB, 1), jnp.int32)
    for it in range(8):
        shift = 28 - 4 * it
        inds = jnp.zeros((RB, 1), jnp.int32)
        for v in range(1, 16):
            c = v << shift
            if c >= 1 << 31:
                c -= 1 << 32
            test = p + jnp.int32(c)
            cnt = jnp.sum((key >= (test ^ int_min)).astype(jnp.int32),
                          axis=1, keepdims=True)
            inds = inds + (cnt >= TOPK).astype(jnp.int32)
        p = p + (inds << shift)
    selected = key >= (p ^ int_min)

    qp = _bdot(lnq, wqt_ref[...]) + bq_ref[...]

    pr = lax.broadcasted_iota(jnp.int32, (4 * RB, RB), 0)
    pc = lax.broadcasted_iota(jnp.int32, (4 * RB, RB), 1)
    psel = _b16((pr // NH == pc).astype(jnp.float32))
    hr = lax.broadcasted_iota(jnp.int32, (4 * RB, D), 0)
    hc = lax.broadcasted_iota(jnp.int32, (4 * RB, D), 1)
    hmask = (hc // HD == hr % NH).astype(jnp.float32)
    p2r = lax.broadcasted_iota(jnp.int32, (RB, 4 * RB), 0)
    p2c = lax.broadcasted_iota(jnp.int32, (RB, 4 * RB), 1)
    p2 = _b16((p2c // NH == p2r).astype(jnp.float32))

    qh32 = _bdot(psel, qp) * hmask
    sel32 = _bdot(psel, selected.astype(jnp.float32)) > 0.5

    scale = 1.0 / jnp.sqrt(jnp.float32(HD))
    s32 = jnp.zeros((4 * RB, UP), jnp.float32)
    for j in range(RB):
        sj = _bdot_t(qh32, c_refs[j][:, 2 * D:3 * D])
        s32 = s32 + sj * (row32 // NH == j).astype(jnp.float32)
    s32 = s32 * scale
    s32 = jnp.where(sel32, s32, -1e30)
    mx = jnp.max(s32, axis=1, keepdims=True)
    e32 = jnp.where(sel32, jnp.exp(s32 - mx), 0.0)
    attn32 = e32 / jnp.sum(e32, axis=1, keepdims=True)

    ctx32 = jnp.zeros((4 * RB, D), jnp.float32)
    for j in range(RB):
        aj = attn32 * (row32 // NH == j).astype(jnp.float32)
        ctx32 = ctx32 + _bdot(aj, c_refs[j][:, 3 * D:4 * D])
    ctx = _bdot(p2, ctx32 * hmask)

    out_ref[...] = _bdot(ctx, owt_ref[...]) + ob_ref[...]


def kernel(patches, patch_ids, valid_mask, patch_center_gps, offsets,
           W1, b1, W2, b2, lnq_g, lnq_b, lnk_g, lnk_b, in_w, in_b, out_w, out_b):
    N, u, Din = patches.shape
    M = patch_ids.shape[0]
    hid = W1.shape[0]

    hg = int(u ** 0.5)
    dx = offsets[:, 0]
    dy = offsets[:, 1]
    i_t = jnp.clip(hg // 2 + dy, 0, hg - 1)
    j_t = jnp.clip(hg // 2 + dx, 0, hg - 1)
    idx_flat = (i_t * hg + j_t).astype(jnp.int32)
    ids = patch_ids.astype(jnp.int32)

    wq, wk, wv = in_w[:D], in_w[D:2 * D], in_w[2 * D:]
    bq, bk, bv = in_b[:D], in_b[D:2 * D], in_b[2 * D:]
    row2 = lambda v: v.reshape(1, -1)

    full2 = lambda r, c: pl.BlockSpec((r, c), lambda t: (0, 0))
    c_all = pl.pallas_call(
        _k1_body,
        grid=(N // PB,),
        in_specs=[
            pl.BlockSpec((PB, u, Din), lambda t: (t, 0, 0)),
            full2(Din, hid), full2(1, hid),
            full2(hid, D), full2(1, D),
            full2(1, D), full2(1, D), full2(1, D), full2(1, D),
            full2(D, D), full2(1, D),
            full2(D, D), full2(1, D),
        ],
        out_specs=pl.BlockSpec((PB * UP, 4 * D), lambda t: (t, 0)),
        out_shape=jax.ShapeDtypeStruct((N * UP, 4 * D), jnp.bfloat16),
    )(patches, W1.T, row2(b1), W2.T, row2(b2),
      row2(lnk_g), row2(lnk_b), row2(lnq_g), row2(lnq_b),
      wk.T, row2(bk), wv.T, row2(bv))

    def gat(j):
        return pl.BlockSpec(
            (UP, 4 * D), lambda m, ids_r, idx_r, j=j: (ids_r[m * RB + j], 0))
    cst = lambda r, c: pl.BlockSpec((r, c), lambda m, ids_r, idx_r: (0, 0))
    grid_spec = pltpu.PrefetchScalarGridSpec(
        num_scalar_prefetch=2,
        grid=(M // RB,),
        in_specs=(
            [pl.BlockSpec((RB, 1), lambda m, ids_r, idx_r: (m, 0))] +
            [gat(j) for j in range(RB)] +
            [cst(D, D), cst(1, D), cst(D, D), cst(1, D)]
        ),
        out_specs=pl.BlockSpec((RB, D), lambda m, ids_r, idx_r: (m, 0)),
    )
    attn_out = pl.pallas_call(
        _k2_body,
        grid_spec=grid_spec,
        out_shape=jax.ShapeDtypeStruct((M, D), jnp.float32),
    )(ids, idx_flat, idx_flat.reshape(M, 1), *([c_all] * RB),
      wq.T, row2(bq), out_w.T, row2(out_b))

    B, T = valid_mask.shape
    flat_mask = valid_mask.reshape(-1)
    rank = jnp.cumsum(flat_mask.astype(jnp.int32)) - 1
    placed = attn_out[jnp.clip(rank, 0, M - 1)]
    return jnp.where(flat_mask[:, None], placed,
                     jnp.zeros((), dtype=attn_out.dtype)).reshape(B, T, D)


# speedup vs baseline: 1.3042x; 1.0264x over previous
import jax
import jax.numpy as jnp
from jax import lax
from jax.experimental import pallas as pl
from jax.experimental.pallas import tpu as pltpu

U = 196
UP = 208
TOPK = 64
NH = 4
HD = 64
D = NH * HD
PB = 16
RB = 32


def _b16(x):
    return x.astype(jnp.bfloat16)


def _bdot(x, y):
    return jnp.dot(_b16(x), _b16(y), preferred_element_type=jnp.float32)


def _bdot_t(x, y):
    return lax.dot_general(_b16(x), _b16(y), (((1,), (1,)), ((), ())),
                           preferred_element_type=jnp.float32)


def _bdot3(x, y):
    return lax.dot_general(_b16(x), _b16(y), (((2,), (1,)), ((), ())),
                           preferred_element_type=jnp.float32)


def _k1_body(x_ref, w1t_ref, b1_ref, w2t_ref, b2_ref,
             lnkg_ref, lnkb_ref, lnqg_ref, lnqb_ref,
             wkt_ref, bk_ref, wvt_ref, bv_ref, c_ref):
    x = x_ref[...]
    h = _bdot3(x, w1t_ref[...]) + b1_ref[...]
    h = jnp.where(h > 0, h, 0.01 * h)
    a = _bdot3(h, w2t_ref[...]) + b2_ref[...]
    anorm = jnp.sqrt(jnp.sum(a * a, axis=2, keepdims=True))
    an = a / jnp.maximum(anorm, 1e-12)
    mu = jnp.mean(a, axis=-1, keepdims=True)
    var = jnp.mean((a - mu) * (a - mu), axis=-1, keepdims=True)
    lnc = (a - mu) / jnp.sqrt(var + 1e-5)
    lnq = lnc * lnqg_ref[...] + lnqb_ref[...]
    lnk = lnc * lnkg_ref[...] + lnkb_ref[...]
    kp = _bdot3(lnk, wkt_ref[...]) + bk_ref[...]
    vp = _bdot3(lnk, wvt_ref[...]) + bv_ref[...]
    packed = jnp.concatenate(
        [_b16(an), _b16(lnq), _b16(kp), _b16(vp)], axis=2)
    for i in range(PB):
        c_ref[i * UP:i * UP + U, :] = packed[i]
        c_ref[i * UP + U:(i + 1) * UP, :] = jnp.zeros(
            (UP - U, 4 * D), jnp.bfloat16)


def _k2_body(ids_ref, idx_sm_ref, idx_ref, *refs):
    c_refs = refs[:RB]
    (wqt_ref, bq_ref, owt_ref, ob_ref, out_ref) = refs[RB:]

    idxv = idx_ref[...]
    rowi = lax.broadcasted_iota(jnp.int32, (RB, 1), 0)
    row32 = lax.broadcasted_iota(jnp.int32, (4 * RB, 1), 0)

    toki = lax.broadcasted_iota(jnp.int32, (RB, UP), 1)
    onehot = _b16((toki == idxv).astype(jnp.float32))

    ql = jnp.zeros((RB, 2 * D), jnp.float32)
    for j in range(RB):
        e = _bdot(onehot, c_refs[j][:, 0:2 * D])
        ql = ql + e * (rowi == j).astype(jnp.float32)
    qn = ql[:, 0:D]
    lnq = ql[:, D:2 * D]

    sim = jnp.zeros((RB, UP), jnp.float32)
    for j in range(RB):
        s = _bdot_t(qn, c_refs[j][:, 0:D])
        sim = sim + s * (rowi == j).astype(jnp.float32)
    col = lax.broadcasted_iota(jnp.int32, (RB, UP), 1)
    simv = jnp.where(col < U, sim, -3.0)

    key = lax.bitcast_convert_type(simv, jnp.int32)
    key = jnp.where(key < 0, key ^ jnp.int32(0x7FFFFFFF), key)
    int_min = jnp.int32(-2147483648)
    p = jnp.zeros((RB, 1), jnp.int32)
    for it in range(8):
        shift = 28 - 4 * it
        inds = jnp.zeros((RB, 1), jnp.int32)
        for v in range(1, 16):
            c = v << shift
            if c >= 1 << 31:
                c -= 1 << 32
            test = p + jnp.int32(c)
            cnt = jnp.sum((key >= (test ^ int_min)).astype(jnp.int32),
                          axis=1, keepdims=True)
            inds = inds + (cnt >= TOPK).astype(jnp.int32)
        p = p + (inds << shift)
    selected = key >= (p ^ int_min)

    qp = _bdot_t(lnq, wqt_ref[...]) + bq_ref[...]

    pr = lax.broadcasted_iota(jnp.int32, (4 * RB, RB), 0)
    pc = lax.broadcasted_iota(jnp.int32, (4 * RB, RB), 1)
    psel = _b16((pr // NH == pc).astype(jnp.float32))
    hr = lax.broadcasted_iota(jnp.int32, (4 * RB, D), 0)
    hc = lax.broadcasted_iota(jnp.int32, (4 * RB, D), 1)
    hmask = (hc // HD == hr % NH).astype(jnp.float32)
    p2r = lax.broadcasted_iota(jnp.int32, (RB, 4 * RB), 0)
    p2c = lax.broadcasted_iota(jnp.int32, (RB, 4 * RB), 1)
    p2 = _b16((p2c // NH == p2r).astype(jnp.float32))

    qh32 = _bdot(psel, qp) * hmask
    sel32 = _bdot(psel, selected.astype(jnp.float32)) > 0.5

    scale = 1.0 / jnp.sqrt(jnp.float32(HD))
    s32 = jnp.zeros((4 * RB, UP), jnp.float32)
    for j in range(RB):
        sj = _bdot_t(qh32, c_refs[j][:, 2 * D:3 * D])
        s32 = s32 + sj * (row32 // NH == j).astype(jnp.float32)
    s32 = s32 * scale
    s32 = jnp.where(sel32, s32, -1e30)
    mx = jnp.max(s32, axis=1, keepdims=True)
    e32 = jnp.where(sel32, jnp.exp(s32 - mx), 0.0)
    attn32 = e32 / jnp.sum(e32, axis=1, keepdims=True)

    ctx32 = jnp.zeros((4 * RB, D), jnp.float32)
    for j in range(RB):
        aj = attn32 * (row32 // NH == j).astype(jnp.float32)
        ctx32 = ctx32 + _bdot(aj, c_refs[j][:, 3 * D:4 * D])
    ctx = _bdot(p2, ctx32 * hmask)

    out_ref[...] = _bdot_t(ctx, owt_ref[...]) + ob_ref[...]


def kernel(patches, patch_ids, valid_mask, patch_center_gps, offsets,
           W1, b1, W2, b2, lnq_g, lnq_b, lnk_g, lnk_b, in_w, in_b, out_w, out_b):
    N, u, Din = patches.shape
    M = patch_ids.shape[0]
    hid = W1.shape[0]

    hg = int(u ** 0.5)
    dx = offsets[:, 0]
    dy = offsets[:, 1]
    i_t = jnp.clip(hg // 2 + dy, 0, hg - 1)
    j_t = jnp.clip(hg // 2 + dx, 0, hg - 1)
    idx_flat = (i_t * hg + j_t).astype(jnp.int32)
    ids = patch_ids.astype(jnp.int32)

    wq, wk, wv = in_w[:D], in_w[D:2 * D], in_w[2 * D:]
    bq, bk, bv = in_b[:D], in_b[D:2 * D], in_b[2 * D:]
    row2 = lambda v: v.reshape(1, -1)

    full2 = lambda r, c: pl.BlockSpec((r, c), lambda t: (0, 0))
    c_all = pl.pallas_call(
        _k1_body,
        grid=(N // PB,),
        in_specs=[
            pl.BlockSpec((PB, u, Din), lambda t: (t, 0, 0)),
            full2(hid, Din), full2(1, hid),
            full2(D, hid), full2(1, D),
            full2(1, D), full2(1, D), full2(1, D), full2(1, D),
            full2(D, D), full2(1, D),
            full2(D, D), full2(1, D),
        ],
        out_specs=pl.BlockSpec((PB * UP, 4 * D), lambda t: (t, 0)),
        out_shape=jax.ShapeDtypeStruct((N * UP, 4 * D), jnp.bfloat16),
    )(patches, W1, row2(b1), W2, row2(b2),
      row2(lnk_g), row2(lnk_b), row2(lnq_g), row2(lnq_b),
      wk, row2(bk), wv, row2(bv))

    def gat(j):
        return pl.BlockSpec(
            (UP, 4 * D), lambda m, ids_r, idx_r, j=j: (ids_r[m * RB + j], 0))
    cst = lambda r, c: pl.BlockSpec((r, c), lambda m, ids_r, idx_r: (0, 0))
    grid_spec = pltpu.PrefetchScalarGridSpec(
        num_scalar_prefetch=2,
        grid=(M // RB,),
        in_specs=(
            [pl.BlockSpec((RB, 1), lambda m, ids_r, idx_r: (m, 0))] +
            [gat(j) for j in range(RB)] +
            [cst(D, D), cst(1, D), cst(D, D), cst(1, D)]
        ),
        out_specs=pl.BlockSpec((RB, D), lambda m, ids_r, idx_r: (m, 0)),
    )
    attn_out = pl.pallas_call(
        _k2_body,
        grid_spec=grid_spec,
        out_shape=jax.ShapeDtypeStruct((M, D), jnp.float32),
    )(ids, idx_flat, idx_flat.reshape(M, 1), *([c_all] * RB),
      wq, row2(bq), out_w, row2(out_b))

    B, T = valid_mask.shape
    flat_mask = valid_mask.reshape(-1)
    rank = jnp.cumsum(flat_mask.astype(jnp.int32)) - 1
    placed = attn_out[jnp.clip(rank, 0, M - 1)]
    return jnp.where(flat_mask[:, None], placed,
                     jnp.zeros((), dtype=attn_out.dtype)).reshape(B, T, D)
